# baseline (speedup 1.0000x reference)
import jax
import jax.numpy as jnp
from jax import lax
from jax.experimental import pallas as pl

_N = 1024
_C = 128
_NCHUNK = _N // _C
_ROWS_PER_BLOCK = 256
_ROWS_PER_TILE = 128


def _lane_bit(m):
    return 1 << m if m <= 6 else 1 << (m - 7)


def _bitonic_body(x_ref, o_ref):
    lanes = lax.broadcasted_iota(jnp.int32, (1, _C), 1)
    for g in range(_ROWS_PER_BLOCK // _ROWS_PER_TILE):
        _sort_row_group(x_ref, o_ref, g, lanes)


def _sort_row_group(x_ref, o_ref, g, lanes):
    rs = slice(g * _ROWS_PER_TILE, (g + 1) * _ROWS_PER_TILE)
    chunks = [x_ref[rs, v * _C:(v + 1) * _C] for v in range(_NCHUNK)]

    for mk in range(1, 11):
        k = 1 << mk
        for mj in range(mk - 1, -1, -1):
            if mj < 3:
                jc = 1 << mj
                if mk < 3:
                    asc_mask = None
                elif mk < 10:
                    asc_mask = (lanes & _lane_bit(mk)) == 0
                else:
                    asc_mask = None
                for v in range(_NCHUNK):
                    if v & jc:
                        continue
                    w = v | jc
                    mn = jnp.minimum(chunks[v], chunks[w])
                    mx = jnp.maximum(chunks[v], chunks[w])
                    if asc_mask is None:
                        asc = True if mk == 10 else (v & k) == 0
                        if asc:
                            chunks[v], chunks[w] = mn, mx
                        else:
                            chunks[v], chunks[w] = mx, mn
                    else:
                        chunks[v] = jnp.where(asc_mask, mn, mx)
                        chunks[w] = jnp.where(asc_mask, mx, mn)
            else:
                d = _lane_bit(mj)
                low = (lanes & d) == 0
                if mk == 10:
                    tm = low
                else:
                    tm = ((lanes & _lane_bit(mk)) == 0) == low
                for v in range(_NCHUNK):
                    c = chunks[v]
                    p = jnp.where(low, jnp.roll(c, -d, axis=1),
                                  jnp.roll(c, d, axis=1))
                    mn = jnp.minimum(c, p)
                    mx = jnp.maximum(c, p)
                    chunks[v] = jnp.where(tm, mn, mx)

    for p in range(3):
        d = 1 << p
        lbit = (lanes & d) != 0
        for v in range(_NCHUNK):
            if v & d:
                continue
            w = v | d
            lo, hi = chunks[v], chunks[w]
            chunks[v] = jnp.where(lbit, jnp.roll(hi, d, axis=1), lo)
            chunks[w] = jnp.where(lbit, hi, jnp.roll(lo, -d, axis=1))

    for v in range(_NCHUNK):
        o_ref[rs, v * _C:(v + 1) * _C] = chunks[v]


def kernel(x):
    b, t, n = x.shape
    rows = b * t
    x2 = x.reshape(rows, n)
    grid = rows // _ROWS_PER_BLOCK
    out = pl.pallas_call(
        _bitonic_body,
        out_shape=jax.ShapeDtypeStruct((rows, n), x.dtype),
        grid=(grid,),
        in_specs=[pl.BlockSpec((_ROWS_PER_BLOCK, n), lambda g: (g, 0))],
        out_specs=pl.BlockSpec((_ROWS_PER_BLOCK, n), lambda g: (g, 0)),
    )(x2)
    return out.reshape(b, t, n)


# speedup vs baseline: 3.5741x; 3.5741x over previous
import jax
import jax.numpy as jnp
from jax import lax
from jax.experimental import pallas as pl
from jax.experimental.pallas import tpu as pltpu

_N = 1024
_NSLAB = _N // 8
_ROWS_PER_BLOCK = 128


def _bitonic_body(x_ref, o_ref):
    a = x_ref[...]
    t = a.T
    slabs = [t[8 * g:8 * (g + 1), :] for g in range(_NSLAB)]
    subl = lax.broadcasted_iota(jnp.int32, (8, _ROWS_PER_BLOCK), 0)

    for mk in range(1, 11):
        for mj in range(mk - 1, -1, -1):
            if mj <= 6:
                jg = 1 << mj
                if 7 <= mk <= 9:
                    asc_mask = (subl & (1 << (mk - 7))) == 0
                else:
                    asc_mask = None
                for g in range(_NSLAB):
                    if g & jg:
                        continue
                    w = g | jg
                    mn = jnp.minimum(slabs[g], slabs[w])
                    mx = jnp.maximum(slabs[g], slabs[w])
                    if asc_mask is None:
                        asc = True if mk == 10 else (g & (1 << mk)) == 0
                        if asc:
                            slabs[g], slabs[w] = mn, mx
                        else:
                            slabs[g], slabs[w] = mx, mn
                    else:
                        slabs[g] = jnp.where(asc_mask, mn, mx)
                        slabs[w] = jnp.where(asc_mask, mx, mn)
            else:
                d = 1 << (mj - 7)
                low = (subl & d) == 0
                if mk == 10:
                    tm = low
                else:
                    tm = ((subl & (1 << (mk - 7))) == 0) == low
                for g in range(_NSLAB):
                    c = slabs[g]
                    if d == 4:
                        p = jnp.roll(c, d, axis=0)
                    else:
                        p = jnp.where(low, jnp.roll(c, -d, axis=0),
                                      jnp.roll(c, d, axis=0))
                    mn = jnp.minimum(c, p)
                    mx = jnp.maximum(c, p)
                    slabs[g] = jnp.where(tm, mn, mx)

    for p in range(3):
        d = 1 << p
        sbit = (subl & d) != 0
        for g in range(_NSLAB):
            if g & d:
                continue
            w = g | d
            lo, hi = slabs[g], slabs[w]
            slabs[g] = jnp.where(sbit, jnp.roll(hi, d, axis=0), lo)
            slabs[w] = jnp.where(sbit, hi, jnp.roll(lo, -d, axis=0))

    ordered = [None] * _NSLAB
    for g in range(_NSLAB):
        ordered[(g >> 3) | ((g & 7) << 4)] = slabs[g]
    ts = jnp.concatenate(ordered, axis=0)
    o_ref[...] = ts.T


def kernel(x):
    b, t, n = x.shape
    rows = b * t
    x2 = x.reshape(rows, n)
    grid = rows // _ROWS_PER_BLOCK
    out = pl.pallas_call(
        _bitonic_body,
        out_shape=jax.ShapeDtypeStruct((rows, n), x.dtype),
        grid=(grid,),
        in_specs=[pl.BlockSpec((_ROWS_PER_BLOCK, n), lambda g: (g, 0))],
        out_specs=pl.BlockSpec((_ROWS_PER_BLOCK, n), lambda g: (g, 0)),
        compiler_params=pltpu.CompilerParams(
            dimension_semantics=(pltpu.ARBITRARY,)),
    )(x2)
    return out.reshape(b, t, n)


# speedup vs baseline: 3.8184x; 1.0683x over previous
import jax
import jax.numpy as jnp
from jax import lax
from jax.experimental import pallas as pl
from jax.experimental.pallas import tpu as pltpu

_N = 1024
_NSLAB = _N // 8
_ROWS_PER_BLOCK = 128


def _bitonic_body(x_ref, o_ref):
    a = x_ref[...]
    t = a.T
    slabs = [t[8 * g:8 * (g + 1), :] for g in range(_NSLAB)]
    subl = lax.broadcasted_iota(jnp.int32, (8, _ROWS_PER_BLOCK), 0)

    phase_plan = [(half, mk) for half in range(2) for mk in range(1, 7)]
    phase_plan += [(None, mk) for mk in range(7, 11)]
    for half, mk in phase_plan:
        gs = range(_NSLAB) if half is None else range(
            half * (_NSLAB // 2), (half + 1) * (_NSLAB // 2))
        for mj in range(mk - 1, -1, -1):
            if mj == 6 and 7 <= mk <= 9:
                asc_mask = (subl & (1 << (mk - 7))) == 0
                m = list(slabs)
                for mq in (64, 32, 16, 8, 4, 2, 1):
                    for g in range(_NSLAB):
                        if g & mq:
                            continue
                        w = g | mq
                        m[g], m[w] = (jnp.minimum(m[g], m[w]),
                                      jnp.maximum(m[g], m[w]))
                slabs = [jnp.where(asc_mask, m[g], m[_NSLAB - 1 - g])
                         for g in range(_NSLAB)]
                break
            if mj <= 6:
                jg = 1 << mj
                if 7 <= mk <= 9:
                    asc_mask = (subl & (1 << (mk - 7))) == 0
                else:
                    asc_mask = None
                for g in gs:
                    if g & jg:
                        continue
                    w = g | jg
                    mn = jnp.minimum(slabs[g], slabs[w])
                    mx = jnp.maximum(slabs[g], slabs[w])
                    if asc_mask is None:
                        asc = True if mk == 10 else (g & (1 << mk)) == 0
                        if asc:
                            slabs[g], slabs[w] = mn, mx
                        else:
                            slabs[g], slabs[w] = mx, mn
                    else:
                        slabs[g] = jnp.where(asc_mask, mn, mx)
                        slabs[w] = jnp.where(asc_mask, mx, mn)
            else:
                d = 1 << (mj - 7)
                low = (subl & d) == 0
                if mk == 10:
                    tm = low
                else:
                    tm = ((subl & (1 << (mk - 7))) == 0) == low
                for g in gs:
                    c = slabs[g]
                    if d == 4:
                        p = jnp.roll(c, d, axis=0)
                    else:
                        p = jnp.where(low, jnp.roll(c, -d, axis=0),
                                      jnp.roll(c, d, axis=0))
                    mn = jnp.minimum(c, p)
                    mx = jnp.maximum(c, p)
                    slabs[g] = jnp.where(tm, mn, mx)

    for p in range(3):
        d = 1 << p
        sbit = (subl & d) != 0
        for g in range(_NSLAB):
            if g & d:
                continue
            w = g | d
            lo, hi = slabs[g], slabs[w]
            slabs[g] = jnp.where(sbit, jnp.roll(hi, d, axis=0), lo)
            slabs[w] = jnp.where(sbit, hi, jnp.roll(lo, -d, axis=0))

    ordered = [None] * _NSLAB
    for g in range(_NSLAB):
        ordered[(g >> 3) | ((g & 7) << 4)] = slabs[g]
    ts = jnp.concatenate(ordered, axis=0)
    o_ref[...] = ts.T


def kernel(x):
    b, t, n = x.shape
    rows = b * t
    x2 = x.reshape(rows, n)
    grid = rows // _ROWS_PER_BLOCK
    out = pl.pallas_call(
        _bitonic_body,
        out_shape=jax.ShapeDtypeStruct((rows, n), x.dtype),
        grid=(grid,),
        in_specs=[pl.BlockSpec((_ROWS_PER_BLOCK, n), lambda g: (g, 0))],
        out_specs=pl.BlockSpec((_ROWS_PER_BLOCK, n), lambda g: (g, 0)),
        compiler_params=pltpu.CompilerParams(
            dimension_semantics=(pltpu.ARBITRARY,)),
    )(x2)
    return out.reshape(b, t, n)


# speedup vs baseline: 3.9608x; 1.0373x over previous
import jax
import jax.numpy as jnp
from jax import lax
from jax.experimental import pallas as pl
from jax.experimental.pallas import tpu as pltpu

_N = 1024
_NSLAB = _N // 8
_ROWS_PER_BLOCK = 256


def _bitonic_body(x_ref, o_ref):
    a = x_ref[...]
    t = a.T
    slabs = [t[8 * g:8 * (g + 1), :] for g in range(_NSLAB)]
    subl = lax.broadcasted_iota(jnp.int32, (8, _ROWS_PER_BLOCK), 0)

    phase_plan = [(half, mk) for half in range(2) for mk in range(1, 7)]
    phase_plan += [(None, mk) for mk in range(7, 11)]
    for half, mk in phase_plan:
        gs = range(_NSLAB) if half is None else range(
            half * (_NSLAB // 2), (half + 1) * (_NSLAB // 2))
        for mj in range(mk - 1, -1, -1):
            if mj == 6 and 7 <= mk <= 9:
                asc_mask = (subl & (1 << (mk - 7))) == 0
                m = list(slabs)
                for mq in (64, 32, 16, 8, 4, 2, 1):
                    for g in range(_NSLAB):
                        if g & mq:
                            continue
                        w = g | mq
                        m[g], m[w] = (jnp.minimum(m[g], m[w]),
                                      jnp.maximum(m[g], m[w]))
                slabs = [jnp.where(asc_mask, m[g], m[_NSLAB - 1 - g])
                         for g in range(_NSLAB)]
                break
            if mj <= 6:
                jg = 1 << mj
                if 7 <= mk <= 9:
                    asc_mask = (subl & (1 << (mk - 7))) == 0
                else:
                    asc_mask = None
                for g in gs:
                    if g & jg:
                        continue
                    w = g | jg
                    mn = jnp.minimum(slabs[g], slabs[w])
                    mx = jnp.maximum(slabs[g], slabs[w])
                    if asc_mask is None:
                        asc = True if mk == 10 else (g & (1 << mk)) == 0
                        if asc:
                            slabs[g], slabs[w] = mn, mx
                        else:
                            slabs[g], slabs[w] = mx, mn
                    else:
                        slabs[g] = jnp.where(asc_mask, mn, mx)
                        slabs[w] = jnp.where(asc_mask, mx, mn)
            else:
                d = 1 << (mj - 7)
                low = (subl & d) == 0
                if mk == 10:
                    tm = low
                else:
                    tm = ((subl & (1 << (mk - 7))) == 0) == low
                for g in gs:
                    c = slabs[g]
                    if d == 4:
                        p = jnp.roll(c, d, axis=0)
                    else:
                        p = jnp.where(low, jnp.roll(c, -d, axis=0),
                                      jnp.roll(c, d, axis=0))
                    mn = jnp.minimum(c, p)
                    mx = jnp.maximum(c, p)
                    slabs[g] = jnp.where(tm, mn, mx)

    for p in range(3):
        d = 1 << p
        sbit = (subl & d) != 0
        for g in range(_NSLAB):
            if g & d:
                continue
            w = g | d
            lo, hi = slabs[g], slabs[w]
            slabs[g] = jnp.where(sbit, jnp.roll(hi, d, axis=0), lo)
            slabs[w] = jnp.where(sbit, hi, jnp.roll(lo, -d, axis=0))

    ordered = [None] * _NSLAB
    for g in range(_NSLAB):
        ordered[(g >> 3) | ((g & 7) << 4)] = slabs[g]
    ts = jnp.concatenate(ordered, axis=0)
    o_ref[...] = ts.T


def kernel(x):
    b, t, n = x.shape
    rows = b * t
    x2 = x.reshape(rows, n)
    grid = rows // _ROWS_PER_BLOCK
    out = pl.pallas_call(
        _bitonic_body,
        out_shape=jax.ShapeDtypeStruct((rows, n), x.dtype),
        grid=(grid,),
        in_specs=[pl.BlockSpec((_ROWS_PER_BLOCK, n), lambda g: (g, 0))],
        out_specs=pl.BlockSpec((_ROWS_PER_BLOCK, n), lambda g: (g, 0)),
        compiler_params=pltpu.CompilerParams(
            dimension_semantics=(pltpu.ARBITRARY,)),
    )(x2)
    return out.reshape(b, t, n)


# speedup vs baseline: 4.0256x; 1.0163x over previous
import jax
import jax.numpy as jnp
from jax import lax
from jax.experimental import pallas as pl
from jax.experimental.pallas import tpu as pltpu

_N = 1024
_NSLAB = _N // 8
_ROWS_PER_BLOCK = 512


def _bitonic_body(x_ref, o_ref):
    a = x_ref[...]
    t = a.T
    slabs = [t[8 * g:8 * (g + 1), :] for g in range(_NSLAB)]
    subl = lax.broadcasted_iota(jnp.int32, (8, _ROWS_PER_BLOCK), 0)

    phase_plan = [(half, mk) for half in range(2) for mk in range(1, 7)]
    phase_plan += [(None, mk) for mk in range(7, 11)]
    for half, mk in phase_plan:
        gs = range(_NSLAB) if half is None else range(
            half * (_NSLAB // 2), (half + 1) * (_NSLAB // 2))
        for mj in range(mk - 1, -1, -1):
            if mj == 6 and 7 <= mk <= 9:
                asc_mask = (subl & (1 << (mk - 7))) == 0
                m = list(slabs)
                for mq in (64, 32, 16, 8, 4, 2, 1):
                    for g in range(_NSLAB):
                        if g & mq:
                            continue
                        w = g | mq
                        m[g], m[w] = (jnp.minimum(m[g], m[w]),
                                      jnp.maximum(m[g], m[w]))
                slabs = [jnp.where(asc_mask, m[g], m[_NSLAB - 1 - g])
                         for g in range(_NSLAB)]
                break
            if mj <= 6:
                jg = 1 << mj
                if 7 <= mk <= 9:
                    asc_mask = (subl & (1 << (mk - 7))) == 0
                else:
                    asc_mask = None
                for g in gs:
                    if g & jg:
                        continue
                    w = g | jg
                    mn = jnp.minimum(slabs[g], slabs[w])
                    mx = jnp.maximum(slabs[g], slabs[w])
                    if asc_mask is None:
                        asc = True if mk == 10 else (g & (1 << mk)) == 0
                        if asc:
                            slabs[g], slabs[w] = mn, mx
                        else:
                            slabs[g], slabs[w] = mx, mn
                    else:
                        slabs[g] = jnp.where(asc_mask, mn, mx)
                        slabs[w] = jnp.where(asc_mask, mx, mn)
            else:
                d = 1 << (mj - 7)
                low = (subl & d) == 0
                if mk == 10:
                    tm = low
                else:
                    tm = ((subl & (1 << (mk - 7))) == 0) == low
                for g in gs:
                    c = slabs[g]
                    if d == 4:
                        p = jnp.roll(c, d, axis=0)
                    else:
                        p = jnp.where(low, jnp.roll(c, -d, axis=0),
                                      jnp.roll(c, d, axis=0))
                    mn = jnp.minimum(c, p)
                    mx = jnp.maximum(c, p)
                    slabs[g] = jnp.where(tm, mn, mx)

    for p in range(3):
        d = 1 << p
        sbit = (subl & d) != 0
        for g in range(_NSLAB):
            if g & d:
                continue
            w = g | d
            lo, hi = slabs[g], slabs[w]
            slabs[g] = jnp.where(sbit, jnp.roll(hi, d, axis=0), lo)
            slabs[w] = jnp.where(sbit, hi, jnp.roll(lo, -d, axis=0))

    ordered = [None] * _NSLAB
    for g in range(_NSLAB):
        ordered[(g >> 3) | ((g & 7) << 4)] = slabs[g]
    ts = jnp.concatenate(ordered, axis=0)
    o_ref[...] = ts.T


def kernel(x):
    b, t, n = x.shape
    rows = b * t
    x2 = x.reshape(rows, n)
    grid = rows // _ROWS_PER_BLOCK
    out = pl.pallas_call(
        _bitonic_body,
        out_shape=jax.ShapeDtypeStruct((rows, n), x.dtype),
        grid=(grid,),
        in_specs=[pl.BlockSpec((_ROWS_PER_BLOCK, n), lambda g: (g, 0))],
        out_specs=pl.BlockSpec((_ROWS_PER_BLOCK, n), lambda g: (g, 0)),
        compiler_params=pltpu.CompilerParams(
            dimension_semantics=(pltpu.ARBITRARY,)),
    )(x2)
    return out.reshape(b, t, n)


# speedup vs baseline: 4.0546x; 1.0072x over previous
import jax
import jax.numpy as jnp
from jax import lax
from jax.experimental import pallas as pl
from jax.experimental.pallas import tpu as pltpu

_N = 1024
_NSLAB = _N // 8
_ROWS_PER_BLOCK = 1024


def _bitonic_body(x_ref, o_ref):
    a = x_ref[...]
    t = a.T
    slabs = [t[8 * g:8 * (g + 1), :] for g in range(_NSLAB)]
    subl = lax.broadcasted_iota(jnp.int32, (8, _ROWS_PER_BLOCK), 0)

    phase_plan = [(half, mk) for half in range(2) for mk in range(1, 7)]
    phase_plan += [(None, mk) for mk in range(7, 11)]
    for half, mk in phase_plan:
        gs = range(_NSLAB) if half is None else range(
            half * (_NSLAB // 2), (half + 1) * (_NSLAB // 2))
        for mj in range(mk - 1, -1, -1):
            if mj == 6 and 7 <= mk <= 9:
                asc_mask = (subl & (1 << (mk - 7))) == 0
                m = list(slabs)
                for mq in (64, 32, 16, 8, 4, 2, 1):
                    for g in range(_NSLAB):
                        if g & mq:
                            continue
                        w = g | mq
                        m[g], m[w] = (jnp.minimum(m[g], m[w]),
                                      jnp.maximum(m[g], m[w]))
                slabs = [jnp.where(asc_mask, m[g], m[_NSLAB - 1 - g])
                         for g in range(_NSLAB)]
                break
            if mj <= 6:
                jg = 1 << mj
                if 7 <= mk <= 9:
                    asc_mask = (subl & (1 << (mk - 7))) == 0
                else:
                    asc_mask = None
                for g in gs:
                    if g & jg:
                        continue
                    w = g | jg
                    mn = jnp.minimum(slabs[g], slabs[w])
                    mx = jnp.maximum(slabs[g], slabs[w])
                    if asc_mask is None:
                        asc = True if mk == 10 else (g & (1 << mk)) == 0
                        if asc:
                            slabs[g], slabs[w] = mn, mx
                        else:
                            slabs[g], slabs[w] = mx, mn
                    else:
                        slabs[g] = jnp.where(asc_mask, mn, mx)
                        slabs[w] = jnp.where(asc_mask, mx, mn)
            else:
                d = 1 << (mj - 7)
                low = (subl & d) == 0
                if mk == 10:
                    tm = low
                else:
                    tm = ((subl & (1 << (mk - 7))) == 0) == low
                for g in gs:
                    c = slabs[g]
                    if d == 4:
                        p = jnp.roll(c, d, axis=0)
                    else:
                        p = jnp.where(low, jnp.roll(c, -d, axis=0),
                                      jnp.roll(c, d, axis=0))
                    mn = jnp.minimum(c, p)
                    mx = jnp.maximum(c, p)
                    slabs[g] = jnp.where(tm, mn, mx)

    for p in range(3):
        d = 1 << p
        sbit = (subl & d) != 0
        for g in range(_NSLAB):
            if g & d:
                continue
            w = g | d
            lo, hi = slabs[g], slabs[w]
            slabs[g] = jnp.where(sbit, jnp.roll(hi, d, axis=0), lo)
            slabs[w] = jnp.where(sbit, hi, jnp.roll(lo, -d, axis=0))

    ordered = [None] * _NSLAB
    for g in range(_NSLAB):
        ordered[(g >> 3) | ((g & 7) << 4)] = slabs[g]
    ts = jnp.concatenate(ordered, axis=0)
    o_ref[...] = ts.T


def kernel(x):
    b, t, n = x.shape
    rows = b * t
    x2 = x.reshape(rows, n)
    grid = rows // _ROWS_PER_BLOCK
    out = pl.pallas_call(
        _bitonic_body,
        out_shape=jax.ShapeDtypeStruct((rows, n), x.dtype),
        grid=(grid,),
        in_specs=[pl.BlockSpec((_ROWS_PER_BLOCK, n), lambda g: (g, 0))],
        out_specs=pl.BlockSpec((_ROWS_PER_BLOCK, n), lambda g: (g, 0)),
        compiler_params=pltpu.CompilerParams(
            dimension_semantics=(pltpu.ARBITRARY,)),
    )(x2)
    return out.reshape(b, t, n)
